# Initial kernel scaffold; baseline (speedup 1.0000x reference)
import jax
import jax.numpy as jnp
from jax.experimental import pallas as pl


def kernel(image):
    raise NotImplementedError("write your pallas kernel here")


# speedup vs baseline: 2.2052x; 2.2052x over previous
import jax
import jax.numpy as jnp
from jax import lax
from jax.experimental import pallas as pl
from jax.experimental.pallas import tpu as pltpu
from jax.experimental.pallas import tpu_sc as plsc

H = 512
W = 512
HW = H * W
NC = 2
NS = 16
L = 16
NW = NC * NS
PIX = HW // NW
CH = 3
TILE = 128
TPW = PIX // TILE


def _body(img_hbm, coords_hbm, colors_hbm, co_v):
    wid = lax.axis_index("s") * NC + lax.axis_index("c")
    base = wid * PIX

    pltpu.sync_copy(img_hbm.at[pl.ds(base * CH, PIX * CH)],
                    colors_hbm.at[pl.ds(base * CH, PIX * CH)])

    lanes = lax.iota(jnp.int32, L)
    ramps = [j * L + lanes for j in range(TILE // L)]

    def tilegen(tt, carry):
        t = wid * TPW + tt
        y = t >> 2
        x0 = (t & 3) * TILE
        o = tt * (2 * TILE)
        for j in range(TILE // L):
            co_v[pl.ds(o + j * L, L)] = jnp.broadcast_to(y, (L,))
            co_v[pl.ds(o + TILE + j * L, L)] = x0 + ramps[j]
        return carry

    lax.fori_loop(0, TPW, tilegen, 0)

    pltpu.sync_copy(co_v, coords_hbm.at[pl.ds(base * 2, PIX * 2)])


@jax.jit
def kernel(image):
    img_flat = image.reshape(CH * HW)
    mesh = plsc.VectorSubcoreMesh(
        core_axis_name="c", subcore_axis_name="s", num_cores=NC, num_subcores=NS)
    coords_flat, colors_flat = pl.kernel(
        _body,
        out_type=(
            jax.ShapeDtypeStruct((2 * HW,), jnp.int32),
            jax.ShapeDtypeStruct((CH * HW,), jnp.float32),
        ),
        mesh=mesh,
        scratch_types=[
            pltpu.VMEM((2 * PIX,), jnp.int32),
        ],
        compiler_params=pltpu.CompilerParams(needs_layout_passes=False),
    )(img_flat)
    coords = coords_flat.reshape(HW // TILE, 2, TILE).transpose(0, 2, 1)
    colors = colors_flat.reshape(CH, 1, HW).transpose(1, 2, 0)
    return coords.reshape(HW, 2), colors


# speedup vs baseline: 10.5421x; 4.7805x over previous
import jax
import jax.numpy as jnp
from jax import lax
from jax.experimental import pallas as pl
from jax.experimental.pallas import tpu as pltpu
from jax.experimental.pallas import tpu_sc as plsc

H = 512
W = 512
HW = H * W
NC = 2
NS = 16
L = 16
NW = NC * NS
PIX = HW // NW
CH = 3
TILE = 128
TPW = PIX // TILE


def _body(img_hbm, coords_hbm, colors_hbm, co_v, px_v, sem_in, sem_out):
    wid = lax.axis_index("s") * NC + lax.axis_index("c")
    base = wid * PIX

    in_cp = pltpu.async_copy(img_hbm.at[pl.ds(base * CH, PIX * CH)],
                             px_v, sem_in)

    lanes = lax.iota(jnp.int32, L)
    ramps = [j * L + lanes for j in range(TILE // L)]

    def tilegen(tt, carry):
        t = wid * TPW + tt
        y = t >> 2
        x0 = (t & 3) * TILE
        o = tt * (2 * TILE)
        for j in range(TILE // L):
            co_v[pl.ds(o + j * L, L)] = jnp.broadcast_to(y, (L,))
            co_v[pl.ds(o + TILE + j * L, L)] = x0 + ramps[j]
        return carry

    lax.fori_loop(0, TPW, tilegen, 0)

    out_cp = pltpu.async_copy(co_v, coords_hbm.at[pl.ds(base * 2, PIX * 2)],
                              sem_out)
    in_cp.wait()
    pltpu.sync_copy(px_v, colors_hbm.at[pl.ds(base * CH, PIX * CH)])
    out_cp.wait()


@jax.jit
def kernel(image):
    img_flat = image.reshape(CH * HW)
    mesh = plsc.VectorSubcoreMesh(
        core_axis_name="c", subcore_axis_name="s", num_cores=NC, num_subcores=NS)
    coords_flat, colors_flat = pl.kernel(
        _body,
        out_type=(
            jax.ShapeDtypeStruct((2 * HW,), jnp.int32),
            jax.ShapeDtypeStruct((CH * HW,), jnp.float32),
        ),
        mesh=mesh,
        scratch_types=[
            pltpu.VMEM((2 * PIX,), jnp.int32),
            pltpu.VMEM((CH * PIX,), jnp.float32),
            pltpu.SemaphoreType.DMA,
            pltpu.SemaphoreType.DMA,
        ],
        compiler_params=pltpu.CompilerParams(needs_layout_passes=False),
    )(img_flat)
    coords = coords_flat.reshape(HW // TILE, 2, TILE).transpose(0, 2, 1)
    colors = colors_flat.reshape(CH, 1, HW).transpose(1, 2, 0)
    return coords.reshape(HW, 2), colors
